# Initial kernel scaffold; baseline (speedup 1.0000x reference)
import jax
import jax.numpy as jnp
from jax.experimental import pallas as pl


def kernel(xyz, points, W0, b0, g0, be0, W1, b1, g1, be1, W2, b2, g2, be2):
    raise NotImplementedError("write your pallas kernel here")


# speedup vs baseline: 12.0684x; 12.0684x over previous
import functools

import jax
import jax.numpy as jnp
from jax.experimental import pallas as pl
from jax.experimental.pallas import tpu as pltpu
from jax.experimental.pallas import tpu_sc as plsc

BATCH = 16
NPTS = 4096
NCENT = 512
NSAMP = 32
RAD2 = 0.2 * 0.2
NWORK = 32
ROWS_PER_W = BATCH * NCENT // NWORK
SHALF = NCENT // 2
NCHUNK = NPTS // 16
FLAT = NCENT * NSAMP


def _fps_body(xyz_ref, out_ref):
    x = xyz_ref[:, 0, :]
    y = xyz_ref[:, 1, :]
    z = xyz_ref[:, 2, :]
    iota = jax.lax.broadcasted_iota(jnp.int32, (BATCH, NPTS), 1)
    iota_c = jax.lax.broadcasted_iota(jnp.int32, (BATCH, NCENT), 1)

    def step(i, carry):
        dist, far, ax, ay, az = carry
        sel = iota == far
        cx = jnp.sum(jnp.where(sel, x, 0.0), axis=1, keepdims=True)
        cy = jnp.sum(jnp.where(sel, y, 0.0), axis=1, keepdims=True)
        cz = jnp.sum(jnp.where(sel, z, 0.0), axis=1, keepdims=True)
        here = iota_c == i
        ax = jnp.where(here, cx, ax)
        ay = jnp.where(here, cy, ay)
        az = jnp.where(here, cz, az)
        dx = x - cx
        dy = y - cy
        dz = z - cz
        d = dx * dx + dy * dy + dz * dz
        dist = jnp.minimum(dist, d)
        m = jnp.max(dist, axis=1, keepdims=True)
        far = jnp.min(jnp.where(dist == m, iota, NPTS), axis=1, keepdims=True)
        return dist, far, ax, ay, az

    zc = jnp.zeros((BATCH, NCENT), jnp.float32)
    init = (jnp.full((BATCH, NPTS), 1e10, jnp.float32),
            jnp.zeros((BATCH, 1), jnp.int32), zc, zc, zc)
    _, _, ax, ay, az = jax.lax.fori_loop(0, NCENT, step, init)
    out_ref[:, 0, :] = ax
    out_ref[:, 1, :] = ay
    out_ref[:, 2, :] = az


def _fps(xyz):
    return pl.pallas_call(
        _fps_body,
        out_shape=jax.ShapeDtypeStruct((BATCH, 3, NCENT), jnp.float32),
    )(xyz)


def _round_bf16(v):
    u = plsc.bitcast(v, jnp.uint32)
    r = (u + jnp.uint32(0x7FFF) + ((u >> jnp.uint32(16)) & jnp.uint32(1)))
    r = r & jnp.uint32(0xFFFF0000)
    return plsc.bitcast(r, jnp.float32)


def _group_body(xyz_hbm, pts_hbm, nxyz_hbm, out_hbm,
                xyz_v, pts_v, sq_v, bxyz_v, cent_v, idx_v, obuf_v):
    wid = jax.lax.axis_index("s") * 2 + jax.lax.axis_index("c")
    b = wid // 2
    s0 = (wid % 2) * SHALF

    pltpu.sync_copy(xyz_hbm.at[b], xyz_v)
    pltpu.sync_copy(pts_hbm.at[b], pts_v)
    for c in range(3):
        pltpu.sync_copy(nxyz_hbm.at[pl.ds((b * 3 + c) * NCENT + s0, SHALF)],
                        cent_v.at[pl.ds(c * SHALF, SHALF)])

    lane = jax.lax.iota(jnp.int32, 16)
    zeros16 = jnp.zeros((16,), jnp.int32)
    ones16 = jnp.full((16,), 1, jnp.int32)
    twos16 = jnp.full((16,), 2, jnp.int32)

    def sq_step(ci, _):
        xv = xyz_v[0, pl.ds(ci * 16, 16)]
        yv = xyz_v[1, pl.ds(ci * 16, 16)]
        zv = xyz_v[2, pl.ds(ci * 16, 16)]
        sq_v[pl.ds(ci * 16, 16)] = (xv * xv + yv * yv) + zv * zv
        bxyz_v[0, pl.ds(ci * 16, 16)] = _round_bf16(xv)
        bxyz_v[1, pl.ds(ci * 16, 16)] = _round_bf16(yv)
        bxyz_v[2, pl.ds(ci * 16, 16)] = _round_bf16(zv)
        return 0

    jax.lax.fori_loop(0, NCHUNK, sq_step, 0)

    def row_body(r, _):
        rv = jnp.full((16,), r, jnp.int32)
        cx = plsc.load_gather(cent_v, [rv])
        cy = plsc.load_gather(cent_v, [rv + SHALF])
        cz = plsc.load_gather(cent_v, [rv + 2 * SHALF])
        s2 = (cx * cx + cy * cy) + cz * cz
        bcx = _round_bf16(cx)
        bcy = _round_bf16(cy)
        bcz = _round_bf16(cz)

        def cond(st):
            ci, found = st
            return jnp.logical_and(found < NSAMP, ci < NCHUNK)

        def body(st):
            ci, found = st
            base = ci * 16
            xv = bxyz_v[0, pl.ds(base, 16)]
            yv = bxyz_v[1, pl.ds(base, 16)]
            zv = bxyz_v[2, pl.ds(base, 16)]
            t = xv * bcx + yv * bcy
            t = t + zv * bcz
            d = (-2.0) * t + s2
            d = d + sq_v[pl.ds(base, 16)]
            mask = d <= RAD2
            mi = mask.astype(jnp.int32)
            cnt = jnp.sum(mi)
            gidx = base + lane

            @pl.when(jnp.logical_and(found == 0, cnt > 0))
            def _prefill():
                first = jnp.min(jnp.where(mask, gidx, NPTS))
                fill = jnp.full((16,), first, jnp.int32)
                idx_v[pl.ds(0, 16)] = fill
                idx_v[pl.ds(16, 16)] = fill

            pos = found + jnp.cumsum(mi) - 1
            wmask = jnp.logical_and(mask, pos < NSAMP)
            plsc.store_scatter(idx_v, [pos], gidx, mask=wmask)
            return ci + 1, found + cnt

        jax.lax.while_loop(cond, body, (jnp.int32(0), jnp.int32(0)))

        plane = SHALF * NSAMP
        for half in range(2):
            iv = idx_v[pl.ds(half * 16, 16)]
            col = r * NSAMP + half * 16
            gx = plsc.load_gather(xyz_v, [zeros16, iv])
            gy = plsc.load_gather(xyz_v, [ones16, iv])
            gz = plsc.load_gather(xyz_v, [twos16, iv])
            obuf_v[pl.ds(col, 16)] = gx - cx
            obuf_v[pl.ds(plane + col, 16)] = gy - cy
            obuf_v[pl.ds(2 * plane + col, 16)] = gz - cz
            obuf_v[pl.ds(3 * plane + col, 16)] = plsc.load_gather(
                pts_v, [zeros16, iv])
            obuf_v[pl.ds(4 * plane + col, 16)] = plsc.load_gather(
                pts_v, [ones16, iv])
            obuf_v[pl.ds(5 * plane + col, 16)] = plsc.load_gather(
                pts_v, [twos16, iv])
        return 0

    jax.lax.fori_loop(0, SHALF, row_body, 0)

    plane = SHALF * NSAMP
    for c in range(6):
        pltpu.sync_copy(
            obuf_v.at[pl.ds(c * plane, plane)],
            out_hbm.at[pl.ds((b * 6 + c) * FLAT + s0 * NSAMP, plane)])


def _group(xyz, points, new_xyz):
    mesh = plsc.VectorSubcoreMesh(core_axis_name="c", subcore_axis_name="s")
    f = functools.partial(
        pl.kernel,
        mesh=mesh,
        compiler_params=pltpu.CompilerParams(needs_layout_passes=False),
        out_type=jax.ShapeDtypeStruct((BATCH * 6 * FLAT,), jnp.float32),
        scratch_types=[
            pltpu.VMEM((3, NPTS), jnp.float32),
            pltpu.VMEM((3, NPTS), jnp.float32),
            pltpu.VMEM((NPTS,), jnp.float32),
            pltpu.VMEM((3, NPTS), jnp.float32),
            pltpu.VMEM((3 * SHALF,), jnp.float32),
            pltpu.VMEM((NSAMP,), jnp.int32),
            pltpu.VMEM((6 * SHALF * NSAMP,), jnp.float32),
        ],
    )(_group_body)
    return f(xyz, points, new_xyz.reshape(-1)).reshape(BATCH, 6, FLAT)


def _conv_stats_body(x_ref, w_ref, bias_ref, y_ref, st_ref):
    i = pl.program_id(0)
    y = jax.lax.dot_general(w_ref[...], x_ref[0],
                            (((1,), (0,)), ((), ())),
                            preferred_element_type=jnp.float32)
    y = y + bias_ref[...].reshape(-1, 1)
    y_ref[0] = y
    s1 = jnp.sum(y, axis=1)
    s2 = jnp.sum(y * y, axis=1)
    st = jnp.stack([s1, s2])

    @pl.when(i == 0)
    def _():
        st_ref[...] = st

    @pl.when(i > 0)
    def _():
        st_ref[...] = st_ref[...] + st


def _conv_stats(x, w, bias, cin, cout):
    return pl.pallas_call(
        _conv_stats_body,
        grid=(BATCH,),
        in_specs=[
            pl.BlockSpec((1, cin, FLAT), lambda i: (i, 0, 0)),
            pl.BlockSpec((cout, cin), lambda i: (0, 0)),
            pl.BlockSpec((1, cout), lambda i: (0, 0)),
        ],
        out_specs=[
            pl.BlockSpec((1, cout, FLAT), lambda i: (i, 0, 0)),
            pl.BlockSpec((2, cout), lambda i: (0, 0)),
        ],
        out_shape=[
            jax.ShapeDtypeStruct((BATCH, cout, FLAT), jnp.float32),
            jax.ShapeDtypeStruct((2, cout), jnp.float32),
        ],
    )(x, w, bias.reshape(1, -1))


def _bn_params(st, g, be):
    n = float(BATCH * FLAT)
    mu = st[0] / n
    var = st[1] / n - mu * mu
    scale = g / jnp.sqrt(var + 1e-5)
    shift = be - mu * scale
    return scale, shift


def _bn_conv_stats_body(y_ref, st_ref, g_ref, be_ref, w_ref, bias_ref,
                        y2_ref, st2_ref):
    i = pl.program_id(0)
    st = st_ref[...]
    n = float(BATCH * FLAT)
    mu = st[0] / n
    var = st[1] / n - mu * mu
    scale = (g_ref[...] / jnp.sqrt(var + 1e-5)).reshape(-1, 1)
    shift = be_ref[...].reshape(-1, 1) - mu.reshape(-1, 1) * scale
    h = jnp.maximum(y_ref[0] * scale + shift, 0.0)
    y2 = jax.lax.dot_general(w_ref[...], h,
                             (((1,), (0,)), ((), ())),
                             preferred_element_type=jnp.float32)
    y2 = y2 + bias_ref[...].reshape(-1, 1)
    y2_ref[0] = y2
    s1 = jnp.sum(y2, axis=1)
    s2 = jnp.sum(y2 * y2, axis=1)
    stn = jnp.stack([s1, s2])

    @pl.when(i == 0)
    def _():
        st2_ref[...] = stn

    @pl.when(i > 0)
    def _():
        st2_ref[...] = st2_ref[...] + stn


def _bn_conv_stats(y, st, g, be, w, bias, cin, cout):
    return pl.pallas_call(
        _bn_conv_stats_body,
        grid=(BATCH,),
        in_specs=[
            pl.BlockSpec((1, cin, FLAT), lambda i: (i, 0, 0)),
            pl.BlockSpec((2, cin), lambda i: (0, 0)),
            pl.BlockSpec((1, cin), lambda i: (0, 0)),
            pl.BlockSpec((1, cin), lambda i: (0, 0)),
            pl.BlockSpec((cout, cin), lambda i: (0, 0)),
            pl.BlockSpec((1, cout), lambda i: (0, 0)),
        ],
        out_specs=[
            pl.BlockSpec((1, cout, FLAT), lambda i: (i, 0, 0)),
            pl.BlockSpec((2, cout), lambda i: (0, 0)),
        ],
        out_shape=[
            jax.ShapeDtypeStruct((BATCH, cout, FLAT), jnp.float32),
            jax.ShapeDtypeStruct((2, cout), jnp.float32),
        ],
    )(y, st, g.reshape(1, -1), be.reshape(1, -1), w, bias.reshape(1, -1))


def _bn_max_body(y_ref, st_ref, g_ref, be_ref, out_ref):
    st = st_ref[0]
    n = float(BATCH * FLAT)
    mu = st[0] / n
    var = st[1] / n - mu * mu
    scale = (g_ref[0] / jnp.sqrt(var + 1e-5)).reshape(-1, 1)
    shift = be_ref[0].reshape(-1, 1) - mu.reshape(-1, 1) * scale
    h = jnp.maximum(y_ref[0] * scale + shift, 0.0)
    out_ref[0] = jnp.max(h.reshape(-1, NCENT, NSAMP), axis=2)


def _bn_max(y, st, g, be, cin):
    cb = 32
    ng = cin // cb
    return pl.pallas_call(
        _bn_max_body,
        grid=(BATCH, ng),
        in_specs=[
            pl.BlockSpec((1, cb, FLAT), lambda i, j: (i, j, 0)),
            pl.BlockSpec((1, 2, cb), lambda i, j: (j, 0, 0)),
            pl.BlockSpec((1, 1, cb), lambda i, j: (j, 0, 0)),
            pl.BlockSpec((1, 1, cb), lambda i, j: (j, 0, 0)),
        ],
        out_specs=pl.BlockSpec((1, cb, NCENT), lambda i, j: (i, j, 0)),
        out_shape=jax.ShapeDtypeStruct((BATCH, cin, NCENT), jnp.float32),
    )(y, st.reshape(2, ng, cb).transpose(1, 0, 2),
      g.reshape(ng, 1, cb), be.reshape(ng, 1, cb))


def kernel(xyz, points, W0, b0, g0, be0, W1, b1, g1, be1, W2, b2, g2, be2):
    new_xyz = _fps(xyz)
    grouped = _group(xyz, points, new_xyz)
    y1, st1 = _conv_stats(grouped, W0, b0, 6, 64)
    y2, st2 = _bn_conv_stats(y1, st1, g0, be0, W1, b1, 64, 64)
    y3, st3 = _bn_conv_stats(y2, st2, g1, be1, W2, b2, 64, 128)
    x = _bn_max(y3, st3, g2, be2, 128)
    return (new_xyz, x)


# speedup vs baseline: 20.7493x; 1.7193x over previous
import functools

import jax
import jax.numpy as jnp
from jax.experimental import pallas as pl
from jax.experimental.pallas import tpu as pltpu
from jax.experimental.pallas import tpu_sc as plsc

BATCH = 16
NPTS = 4096
NCENT = 512
NSAMP = 32
RAD2 = 0.2 * 0.2
NWORK = 32
ROWS_PER_W = BATCH * NCENT // NWORK
SHALF = NCENT // 2
NCHUNK = NPTS // 16
FLAT = NCENT * NSAMP


def _fps_body(xyz_ref, out_ref):
    x = xyz_ref[:, 0, :]
    y = xyz_ref[:, 1, :]
    z = xyz_ref[:, 2, :]
    iota = jax.lax.broadcasted_iota(jnp.int32, (BATCH, NPTS), 1)
    iota_c = jax.lax.broadcasted_iota(jnp.int32, (BATCH, NCENT), 1)

    def step(i, carry):
        dist, far, ax, ay, az = carry
        sel = iota == far
        cx = jnp.sum(jnp.where(sel, x, 0.0), axis=1, keepdims=True)
        cy = jnp.sum(jnp.where(sel, y, 0.0), axis=1, keepdims=True)
        cz = jnp.sum(jnp.where(sel, z, 0.0), axis=1, keepdims=True)
        here = iota_c == i
        ax = jnp.where(here, cx, ax)
        ay = jnp.where(here, cy, ay)
        az = jnp.where(here, cz, az)
        dx = x - cx
        dy = y - cy
        dz = z - cz
        d = dx * dx + dy * dy + dz * dz
        dist = jnp.minimum(dist, d)
        m = jnp.max(dist, axis=1, keepdims=True)
        far = jnp.min(jnp.where(dist == m, iota, NPTS), axis=1, keepdims=True)
        return dist, far, ax, ay, az

    zc = jnp.zeros((BATCH, NCENT), jnp.float32)
    init = (jnp.full((BATCH, NPTS), 1e10, jnp.float32),
            jnp.zeros((BATCH, 1), jnp.int32), zc, zc, zc)
    _, _, ax, ay, az = jax.lax.fori_loop(0, NCENT, step, init)
    out_ref[:, 0, :] = ax
    out_ref[:, 1, :] = ay
    out_ref[:, 2, :] = az


def _fps(xyz):
    return pl.pallas_call(
        _fps_body,
        out_shape=jax.ShapeDtypeStruct((BATCH, 3, NCENT), jnp.float32),
    )(xyz)


def _round_bf16(v):
    u = plsc.bitcast(v, jnp.uint32)
    r = (u + jnp.uint32(0x7FFF) + ((u >> jnp.uint32(16)) & jnp.uint32(1)))
    r = r & jnp.uint32(0xFFFF0000)
    return plsc.bitcast(r, jnp.float32)


def _group_body(xyz_hbm, pts_hbm, nxyz_hbm, out_hbm,
                xyz_v, pts_v, sq_v, bxyz_v, cent_v, idx_v, obuf_v):
    wid = jax.lax.axis_index("s") * 2 + jax.lax.axis_index("c")
    b = wid // 2
    s0 = (wid % 2) * SHALF

    pltpu.sync_copy(xyz_hbm.at[b], xyz_v)
    pltpu.sync_copy(pts_hbm.at[b], pts_v)
    for c in range(3):
        pltpu.sync_copy(nxyz_hbm.at[pl.ds((b * 3 + c) * NCENT + s0, SHALF)],
                        cent_v.at[pl.ds(c * SHALF, SHALF)])

    lane = jax.lax.iota(jnp.int32, 16)
    zeros16 = jnp.zeros((16,), jnp.int32)
    ones16 = jnp.full((16,), 1, jnp.int32)
    twos16 = jnp.full((16,), 2, jnp.int32)

    def sq_step(ci, _):
        xv = xyz_v[0, pl.ds(ci * 16, 16)]
        yv = xyz_v[1, pl.ds(ci * 16, 16)]
        zv = xyz_v[2, pl.ds(ci * 16, 16)]
        sq_v[pl.ds(ci * 16, 16)] = (xv * xv + yv * yv) + zv * zv
        bxyz_v[0, pl.ds(ci * 16, 16)] = _round_bf16(xv)
        bxyz_v[1, pl.ds(ci * 16, 16)] = _round_bf16(yv)
        bxyz_v[2, pl.ds(ci * 16, 16)] = _round_bf16(zv)
        return 0

    jax.lax.fori_loop(0, NCHUNK, sq_step, 0)

    def row_body(r, _):
        rv = jnp.full((16,), r, jnp.int32)
        cx = plsc.load_gather(cent_v, [rv])
        cy = plsc.load_gather(cent_v, [rv + SHALF])
        cz = plsc.load_gather(cent_v, [rv + 2 * SHALF])
        s2 = (cx * cx + cy * cy) + cz * cz
        bcx = _round_bf16(cx)
        bcy = _round_bf16(cy)
        bcz = _round_bf16(cz)

        def cond(st):
            g, found = st
            return jnp.logical_and(found < NSAMP, g < NCHUNK // 4)

        def body(st):
            g, found = st
            masks, csums, pcnts, gidxs = [], [], [], []
            tot = None
            for k in range(4):
                base = g * 64 + k * 16
                xv = bxyz_v[0, pl.ds(base, 16)]
                yv = bxyz_v[1, pl.ds(base, 16)]
                zv = bxyz_v[2, pl.ds(base, 16)]
                t = xv * bcx + yv * bcy
                t = t + zv * bcz
                d = (-2.0) * t + s2
                d = d + sq_v[pl.ds(base, 16)]
                mask = d <= RAD2
                mi = mask.astype(jnp.int32)
                masks.append(mask)
                csums.append(jnp.cumsum(mi))
                pcnts.append(plsc.all_reduce_population_count(mask))
                gidxs.append(base + lane)
                tot = mi if tot is None else tot + mi
            total = jnp.sum(tot)

            @pl.when(jnp.logical_and(found == 0, total > 0))
            def _prefill():
                f0 = jnp.minimum(jnp.where(masks[0], gidxs[0], NPTS),
                                 jnp.where(masks[1], gidxs[1], NPTS))
                f1 = jnp.minimum(jnp.where(masks[2], gidxs[2], NPTS),
                                 jnp.where(masks[3], gidxs[3], NPTS))
                first = jnp.min(jnp.minimum(f0, f1))
                fill = jnp.full((16,), first, jnp.int32)
                idx_v[pl.ds(0, 16)] = fill
                idx_v[pl.ds(16, 16)] = fill

            found_vv = jnp.full((16,), found, jnp.int32)
            off = found_vv
            for k in range(4):
                pos = off + csums[k] - 1
                wmask = jnp.logical_and(masks[k], pos < NSAMP)
                plsc.store_scatter(idx_v, [pos], gidxs[k], mask=wmask)
                off = off + pcnts[k]
            return g + 1, found + total

        jax.lax.while_loop(cond, body, (jnp.int32(0), jnp.int32(0)))

        plane = SHALF * NSAMP
        for half in range(2):
            iv = idx_v[pl.ds(half * 16, 16)]
            col = r * NSAMP + half * 16
            gx = plsc.load_gather(xyz_v, [zeros16, iv])
            gy = plsc.load_gather(xyz_v, [ones16, iv])
            gz = plsc.load_gather(xyz_v, [twos16, iv])
            obuf_v[pl.ds(col, 16)] = gx - cx
            obuf_v[pl.ds(plane + col, 16)] = gy - cy
            obuf_v[pl.ds(2 * plane + col, 16)] = gz - cz
            obuf_v[pl.ds(3 * plane + col, 16)] = plsc.load_gather(
                pts_v, [zeros16, iv])
            obuf_v[pl.ds(4 * plane + col, 16)] = plsc.load_gather(
                pts_v, [ones16, iv])
            obuf_v[pl.ds(5 * plane + col, 16)] = plsc.load_gather(
                pts_v, [twos16, iv])
        return 0

    jax.lax.fori_loop(0, SHALF, row_body, 0)

    plane = SHALF * NSAMP
    for c in range(6):
        pltpu.sync_copy(
            obuf_v.at[pl.ds(c * plane, plane)],
            out_hbm.at[pl.ds((b * 6 + c) * FLAT + s0 * NSAMP, plane)])


def _group(xyz, points, new_xyz):
    mesh = plsc.VectorSubcoreMesh(core_axis_name="c", subcore_axis_name="s")
    f = functools.partial(
        pl.kernel,
        mesh=mesh,
        compiler_params=pltpu.CompilerParams(needs_layout_passes=False),
        out_type=jax.ShapeDtypeStruct((BATCH * 6 * FLAT,), jnp.float32),
        scratch_types=[
            pltpu.VMEM((3, NPTS), jnp.float32),
            pltpu.VMEM((3, NPTS), jnp.float32),
            pltpu.VMEM((NPTS,), jnp.float32),
            pltpu.VMEM((3, NPTS), jnp.float32),
            pltpu.VMEM((3 * SHALF,), jnp.float32),
            pltpu.VMEM((NSAMP,), jnp.int32),
            pltpu.VMEM((6 * SHALF * NSAMP,), jnp.float32),
        ],
    )(_group_body)
    return f(xyz, points, new_xyz.reshape(-1)).reshape(BATCH, 6, FLAT)


def _conv_stats_body(x_ref, w_ref, bias_ref, y_ref, st_ref):
    i = pl.program_id(0)
    y = jax.lax.dot_general(w_ref[...], x_ref[0],
                            (((1,), (0,)), ((), ())),
                            preferred_element_type=jnp.float32)
    y = y + bias_ref[...].reshape(-1, 1)
    y_ref[0] = y
    s1 = jnp.sum(y, axis=1)
    s2 = jnp.sum(y * y, axis=1)
    st = jnp.stack([s1, s2])

    @pl.when(i == 0)
    def _():
        st_ref[...] = st

    @pl.when(i > 0)
    def _():
        st_ref[...] = st_ref[...] + st


def _conv_stats(x, w, bias, cin, cout):
    return pl.pallas_call(
        _conv_stats_body,
        grid=(BATCH,),
        in_specs=[
            pl.BlockSpec((1, cin, FLAT), lambda i: (i, 0, 0)),
            pl.BlockSpec((cout, cin), lambda i: (0, 0)),
            pl.BlockSpec((1, cout), lambda i: (0, 0)),
        ],
        out_specs=[
            pl.BlockSpec((1, cout, FLAT), lambda i: (i, 0, 0)),
            pl.BlockSpec((2, cout), lambda i: (0, 0)),
        ],
        out_shape=[
            jax.ShapeDtypeStruct((BATCH, cout, FLAT), jnp.float32),
            jax.ShapeDtypeStruct((2, cout), jnp.float32),
        ],
    )(x, w, bias.reshape(1, -1))


def _bn_params(st, g, be):
    n = float(BATCH * FLAT)
    mu = st[0] / n
    var = st[1] / n - mu * mu
    scale = g / jnp.sqrt(var + 1e-5)
    shift = be - mu * scale
    return scale, shift


def _bn_conv_stats_body(y_ref, st_ref, g_ref, be_ref, w_ref, bias_ref,
                        y2_ref, st2_ref):
    i = pl.program_id(0)
    st = st_ref[...]
    n = float(BATCH * FLAT)
    mu = st[0] / n
    var = st[1] / n - mu * mu
    scale = (g_ref[...] / jnp.sqrt(var + 1e-5)).reshape(-1, 1)
    shift = be_ref[...].reshape(-1, 1) - mu.reshape(-1, 1) * scale
    h = jnp.maximum(y_ref[0] * scale + shift, 0.0)
    y2 = jax.lax.dot_general(w_ref[...], h,
                             (((1,), (0,)), ((), ())),
                             preferred_element_type=jnp.float32)
    y2 = y2 + bias_ref[...].reshape(-1, 1)
    y2_ref[0] = y2
    s1 = jnp.sum(y2, axis=1)
    s2 = jnp.sum(y2 * y2, axis=1)
    stn = jnp.stack([s1, s2])

    @pl.when(i == 0)
    def _():
        st2_ref[...] = stn

    @pl.when(i > 0)
    def _():
        st2_ref[...] = st2_ref[...] + stn


def _bn_conv_stats(y, st, g, be, w, bias, cin, cout):
    return pl.pallas_call(
        _bn_conv_stats_body,
        grid=(BATCH,),
        in_specs=[
            pl.BlockSpec((1, cin, FLAT), lambda i: (i, 0, 0)),
            pl.BlockSpec((2, cin), lambda i: (0, 0)),
            pl.BlockSpec((1, cin), lambda i: (0, 0)),
            pl.BlockSpec((1, cin), lambda i: (0, 0)),
            pl.BlockSpec((cout, cin), lambda i: (0, 0)),
            pl.BlockSpec((1, cout), lambda i: (0, 0)),
        ],
        out_specs=[
            pl.BlockSpec((1, cout, FLAT), lambda i: (i, 0, 0)),
            pl.BlockSpec((2, cout), lambda i: (0, 0)),
        ],
        out_shape=[
            jax.ShapeDtypeStruct((BATCH, cout, FLAT), jnp.float32),
            jax.ShapeDtypeStruct((2, cout), jnp.float32),
        ],
    )(y, st, g.reshape(1, -1), be.reshape(1, -1), w, bias.reshape(1, -1))


def _bn_max_body(y_ref, st_ref, g_ref, be_ref, out_ref):
    st = st_ref[0]
    n = float(BATCH * FLAT)
    mu = st[0] / n
    var = st[1] / n - mu * mu
    scale = (g_ref[0] / jnp.sqrt(var + 1e-5)).reshape(-1, 1)
    shift = be_ref[0].reshape(-1, 1) - mu.reshape(-1, 1) * scale
    h = jnp.maximum(y_ref[0] * scale + shift, 0.0)
    out_ref[0] = jnp.max(h.reshape(-1, NCENT, NSAMP), axis=2)


def _bn_max(y, st, g, be, cin):
    cb = 32
    ng = cin // cb
    return pl.pallas_call(
        _bn_max_body,
        grid=(BATCH, ng),
        in_specs=[
            pl.BlockSpec((1, cb, FLAT), lambda i, j: (i, j, 0)),
            pl.BlockSpec((1, 2, cb), lambda i, j: (j, 0, 0)),
            pl.BlockSpec((1, 1, cb), lambda i, j: (j, 0, 0)),
            pl.BlockSpec((1, 1, cb), lambda i, j: (j, 0, 0)),
        ],
        out_specs=pl.BlockSpec((1, cb, NCENT), lambda i, j: (i, j, 0)),
        out_shape=jax.ShapeDtypeStruct((BATCH, cin, NCENT), jnp.float32),
    )(y, st.reshape(2, ng, cb).transpose(1, 0, 2),
      g.reshape(ng, 1, cb), be.reshape(ng, 1, cb))


def kernel(xyz, points, W0, b0, g0, be0, W1, b1, g1, be1, W2, b2, g2, be2):
    new_xyz = _fps(xyz)
    grouped = _group(xyz, points, new_xyz)
    y1, st1 = _conv_stats(grouped, W0, b0, 6, 64)
    y2, st2 = _bn_conv_stats(y1, st1, g0, be0, W1, b1, 64, 64)
    y3, st3 = _bn_conv_stats(y2, st2, g1, be1, W2, b2, 64, 128)
    x = _bn_max(y3, st3, g2, be2, 128)
    return (new_xyz, x)


# speedup vs baseline: 21.0268x; 1.0134x over previous
import functools

import jax
import jax.numpy as jnp
from jax.experimental import pallas as pl
from jax.experimental.pallas import tpu as pltpu
from jax.experimental.pallas import tpu_sc as plsc

BATCH = 16
NPTS = 4096
NCENT = 512
NSAMP = 32
RAD2 = 0.2 * 0.2
NWORK = 32
ROWS_PER_W = BATCH * NCENT // NWORK
SHALF = NCENT // 2
NCHUNK = NPTS // 16
FLAT = NCENT * NSAMP


def _fps_body(xyz_ref, out_ref):
    x = xyz_ref[:, 0, :]
    y = xyz_ref[:, 1, :]
    z = xyz_ref[:, 2, :]
    iota = jax.lax.broadcasted_iota(jnp.int32, (BATCH, NPTS), 1)
    iota_c = jax.lax.broadcasted_iota(jnp.int32, (BATCH, NCENT), 1)

    def step(i, carry):
        dist, far, ax, ay, az = carry
        sel = iota == far
        cx = jnp.sum(jnp.where(sel, x, 0.0), axis=1, keepdims=True)
        cy = jnp.sum(jnp.where(sel, y, 0.0), axis=1, keepdims=True)
        cz = jnp.sum(jnp.where(sel, z, 0.0), axis=1, keepdims=True)
        here = iota_c == i
        ax = jnp.where(here, cx, ax)
        ay = jnp.where(here, cy, ay)
        az = jnp.where(here, cz, az)
        dx = x - cx
        dy = y - cy
        dz = z - cz
        d = dx * dx + dy * dy + dz * dz
        dist = jnp.minimum(dist, d)
        m = jnp.max(dist, axis=1, keepdims=True)
        far = jnp.min(jnp.where(dist == m, iota, NPTS), axis=1, keepdims=True)
        return dist, far, ax, ay, az

    zc = jnp.zeros((BATCH, NCENT), jnp.float32)
    init = (jnp.full((BATCH, NPTS), 1e10, jnp.float32),
            jnp.zeros((BATCH, 1), jnp.int32), zc, zc, zc)
    _, _, ax, ay, az = jax.lax.fori_loop(0, NCENT, step, init)
    out_ref[:, 0, :] = ax
    out_ref[:, 1, :] = ay
    out_ref[:, 2, :] = az


def _fps(xyz):
    return pl.pallas_call(
        _fps_body,
        out_shape=jax.ShapeDtypeStruct((BATCH, 3, NCENT), jnp.float32),
    )(xyz)


def _round_bf16(v):
    u = plsc.bitcast(v, jnp.uint32)
    r = (u + jnp.uint32(0x7FFF) + ((u >> jnp.uint32(16)) & jnp.uint32(1)))
    r = r & jnp.uint32(0xFFFF0000)
    return plsc.bitcast(r, jnp.float32)


def _group_body(xyz_hbm, pts_hbm, nxyz_hbm, out_hbm,
                xyz_v, pts_v, sq_v, bxyz_v, cent_v, idx_v, obuf_v):
    wid = jax.lax.axis_index("s") * 2 + jax.lax.axis_index("c")
    b = wid // 2
    s0 = (wid % 2) * SHALF

    pltpu.sync_copy(xyz_hbm.at[b], xyz_v)
    pltpu.sync_copy(pts_hbm.at[b], pts_v)
    for c in range(3):
        pltpu.sync_copy(nxyz_hbm.at[pl.ds((b * 3 + c) * NCENT + s0, SHALF)],
                        cent_v.at[pl.ds(c * SHALF, SHALF)])

    lane = jax.lax.iota(jnp.int32, 16)
    zeros16 = jnp.zeros((16,), jnp.int32)
    ones16 = jnp.full((16,), 1, jnp.int32)
    twos16 = jnp.full((16,), 2, jnp.int32)

    def sq_step(ci, _):
        xv = xyz_v[0, pl.ds(ci * 16, 16)]
        yv = xyz_v[1, pl.ds(ci * 16, 16)]
        zv = xyz_v[2, pl.ds(ci * 16, 16)]
        sq_v[pl.ds(ci * 16, 16)] = (xv * xv + yv * yv) + zv * zv
        bxyz_v[0, pl.ds(ci * 16, 16)] = _round_bf16(xv)
        bxyz_v[1, pl.ds(ci * 16, 16)] = _round_bf16(yv)
        bxyz_v[2, pl.ds(ci * 16, 16)] = _round_bf16(zv)
        return 0

    jax.lax.fori_loop(0, NCHUNK, sq_step, 0)

    def row_body(r, _):
        rv = jnp.full((16,), r, jnp.int32)
        cx = plsc.load_gather(cent_v, [rv])
        cy = plsc.load_gather(cent_v, [rv + SHALF])
        cz = plsc.load_gather(cent_v, [rv + 2 * SHALF])
        s2 = (cx * cx + cy * cy) + cz * cz
        bcx = _round_bf16(cx)
        bcy = _round_bf16(cy)
        bcz = _round_bf16(cz)

        def cond(st):
            g, found = st
            return jnp.logical_and(found < NSAMP, g < NCHUNK // 4)

        def body(st):
            g, found = st
            masks, csums, pcnts, gidxs = [], [], [], []
            tot = None
            for k in range(4):
                base = g * 64 + k * 16
                xv = bxyz_v[0, pl.ds(base, 16)]
                yv = bxyz_v[1, pl.ds(base, 16)]
                zv = bxyz_v[2, pl.ds(base, 16)]
                t = xv * bcx + yv * bcy
                t = t + zv * bcz
                d = (-2.0) * t + s2
                d = d + sq_v[pl.ds(base, 16)]
                mask = d <= RAD2
                mi = mask.astype(jnp.int32)
                masks.append(mask)
                csums.append(jnp.cumsum(mi))
                pcnts.append(plsc.all_reduce_population_count(mask))
                gidxs.append(base + lane)
                tot = mi if tot is None else tot + mi
            total = jnp.sum(tot)

            @pl.when(jnp.logical_and(found == 0, total > 0))
            def _prefill():
                f0 = jnp.minimum(jnp.where(masks[0], gidxs[0], NPTS),
                                 jnp.where(masks[1], gidxs[1], NPTS))
                f1 = jnp.minimum(jnp.where(masks[2], gidxs[2], NPTS),
                                 jnp.where(masks[3], gidxs[3], NPTS))
                first = jnp.min(jnp.minimum(f0, f1))
                fill = jnp.full((16,), first, jnp.int32)
                idx_v[pl.ds(0, 16)] = fill
                idx_v[pl.ds(16, 16)] = fill

            found_vv = jnp.full((16,), found, jnp.int32)
            off = found_vv
            for k in range(4):
                pos = off + csums[k] - 1
                wmask = jnp.logical_and(masks[k], pos < NSAMP)
                plsc.store_scatter(idx_v, [pos], gidxs[k], mask=wmask)
                off = off + pcnts[k]
            return g + 1, found + total

        jax.lax.while_loop(cond, body, (jnp.int32(0), jnp.int32(0)))

        plane = SHALF * NSAMP
        for half in range(2):
            iv = idx_v[pl.ds(half * 16, 16)]
            col = r * NSAMP + half * 16
            gx = plsc.load_gather(xyz_v, [zeros16, iv])
            gy = plsc.load_gather(xyz_v, [ones16, iv])
            gz = plsc.load_gather(xyz_v, [twos16, iv])
            obuf_v[pl.ds(col, 16)] = gx - cx
            obuf_v[pl.ds(plane + col, 16)] = gy - cy
            obuf_v[pl.ds(2 * plane + col, 16)] = gz - cz
            obuf_v[pl.ds(3 * plane + col, 16)] = plsc.load_gather(
                pts_v, [zeros16, iv])
            obuf_v[pl.ds(4 * plane + col, 16)] = plsc.load_gather(
                pts_v, [ones16, iv])
            obuf_v[pl.ds(5 * plane + col, 16)] = plsc.load_gather(
                pts_v, [twos16, iv])
        return 0

    jax.lax.fori_loop(0, SHALF, row_body, 0)

    plane = SHALF * NSAMP
    for c in range(6):
        pltpu.sync_copy(
            obuf_v.at[pl.ds(c * plane, plane)],
            out_hbm.at[pl.ds((b * 6 + c) * FLAT + s0 * NSAMP, plane)])


def _group(xyz, points, new_xyz):
    mesh = plsc.VectorSubcoreMesh(core_axis_name="c", subcore_axis_name="s")
    f = functools.partial(
        pl.kernel,
        mesh=mesh,
        compiler_params=pltpu.CompilerParams(needs_layout_passes=False),
        out_type=jax.ShapeDtypeStruct((BATCH * 6 * FLAT,), jnp.float32),
        scratch_types=[
            pltpu.VMEM((3, NPTS), jnp.float32),
            pltpu.VMEM((3, NPTS), jnp.float32),
            pltpu.VMEM((NPTS,), jnp.float32),
            pltpu.VMEM((3, NPTS), jnp.float32),
            pltpu.VMEM((3 * SHALF,), jnp.float32),
            pltpu.VMEM((NSAMP,), jnp.int32),
            pltpu.VMEM((6 * SHALF * NSAMP,), jnp.float32),
        ],
    )(_group_body)
    return f(xyz, points, new_xyz.reshape(-1)).reshape(BATCH, 6, FLAT)


def _conv_stats_body(x_ref, w_ref, bias_ref, y_ref, st_ref):
    i = pl.program_id(0)
    y = jax.lax.dot_general(w_ref[...], x_ref[0],
                            (((1,), (0,)), ((), ())),
                            preferred_element_type=jnp.float32)
    y = y + bias_ref[...].reshape(-1, 1)
    y_ref[0] = y
    s1 = jnp.sum(y, axis=1)
    s2 = jnp.sum(y * y, axis=1)
    st = jnp.stack([s1, s2])

    @pl.when(i == 0)
    def _():
        st_ref[...] = st

    @pl.when(i > 0)
    def _():
        st_ref[...] = st_ref[...] + st


def _conv_stats(x, w, bias, cin, cout):
    return pl.pallas_call(
        _conv_stats_body,
        grid=(BATCH,),
        in_specs=[
            pl.BlockSpec((1, cin, FLAT), lambda i: (i, 0, 0)),
            pl.BlockSpec((cout, cin), lambda i: (0, 0)),
            pl.BlockSpec((1, cout), lambda i: (0, 0)),
        ],
        out_specs=[
            pl.BlockSpec((1, cout, FLAT), lambda i: (i, 0, 0)),
            pl.BlockSpec((2, cout), lambda i: (0, 0)),
        ],
        out_shape=[
            jax.ShapeDtypeStruct((BATCH, cout, FLAT), jnp.float32),
            jax.ShapeDtypeStruct((2, cout), jnp.float32),
        ],
    )(x, w, bias.reshape(1, -1))


def _bn_params(st, g, be):
    n = float(BATCH * FLAT)
    mu = st[0] / n
    var = st[1] / n - mu * mu
    scale = g / jnp.sqrt(var + 1e-5)
    shift = be - mu * scale
    return scale, shift


def _bn_conv_stats_body(y_ref, st_ref, g_ref, be_ref, w_ref, bias_ref,
                        y2_ref, st2_ref):
    i = pl.program_id(0)
    st = st_ref[...]
    n = float(BATCH * FLAT)
    mu = st[0] / n
    var = st[1] / n - mu * mu
    scale = (g_ref[...] / jnp.sqrt(var + 1e-5)).reshape(-1, 1)
    shift = be_ref[...].reshape(-1, 1) - mu.reshape(-1, 1) * scale
    h = jnp.maximum(y_ref[0] * scale + shift, 0.0)
    y2 = jax.lax.dot_general(w_ref[...], h,
                             (((1,), (0,)), ((), ())),
                             preferred_element_type=jnp.float32)
    y2 = y2 + bias_ref[...].reshape(-1, 1)
    y2_ref[0] = y2
    s1 = jnp.sum(y2, axis=1)
    s2 = jnp.sum(y2 * y2, axis=1)
    stn = jnp.stack([s1, s2])

    @pl.when(i == 0)
    def _():
        st2_ref[...] = stn

    @pl.when(i > 0)
    def _():
        st2_ref[...] = st2_ref[...] + stn


def _bn_conv_stats(y, st, g, be, w, bias, cin, cout):
    return pl.pallas_call(
        _bn_conv_stats_body,
        grid=(BATCH,),
        in_specs=[
            pl.BlockSpec((1, cin, FLAT), lambda i: (i, 0, 0)),
            pl.BlockSpec((2, cin), lambda i: (0, 0)),
            pl.BlockSpec((1, cin), lambda i: (0, 0)),
            pl.BlockSpec((1, cin), lambda i: (0, 0)),
            pl.BlockSpec((cout, cin), lambda i: (0, 0)),
            pl.BlockSpec((1, cout), lambda i: (0, 0)),
        ],
        out_specs=[
            pl.BlockSpec((1, cout, FLAT), lambda i: (i, 0, 0)),
            pl.BlockSpec((2, cout), lambda i: (0, 0)),
        ],
        out_shape=[
            jax.ShapeDtypeStruct((BATCH, cout, FLAT), jnp.float32),
            jax.ShapeDtypeStruct((2, cout), jnp.float32),
        ],
    )(y, st, g.reshape(1, -1), be.reshape(1, -1), w, bias.reshape(1, -1))


def _bn_relu_cov_body(y_ref, st_ref, g_ref, be_ref, h_ref, cov_ref):
    i = pl.program_id(0)
    st = st_ref[...]
    n = float(BATCH * FLAT)
    mu = st[0] / n
    var = st[1] / n - mu * mu
    scale = (g_ref[...] / jnp.sqrt(var + 1e-5)).reshape(-1, 1)
    shift = be_ref[...].reshape(-1, 1) - mu.reshape(-1, 1) * scale
    h = jnp.maximum(y_ref[0] * scale + shift, 0.0)
    h_ref[0] = h
    m2 = jax.lax.dot_general(h, h, (((1,), (1,)), ((), ())),
                             preferred_element_type=jnp.float32)
    s = jnp.sum(h, axis=1).reshape(1, -1)
    cov = jnp.concatenate([m2, s], axis=0)

    @pl.when(i == 0)
    def _():
        cov_ref[...] = cov

    @pl.when(i > 0)
    def _():
        cov_ref[...] = cov_ref[...] + cov


def _bn_relu_cov(y, st, g, be, cin):
    return pl.pallas_call(
        _bn_relu_cov_body,
        grid=(BATCH,),
        in_specs=[
            pl.BlockSpec((1, cin, FLAT), lambda i: (i, 0, 0)),
            pl.BlockSpec((2, cin), lambda i: (0, 0)),
            pl.BlockSpec((1, cin), lambda i: (0, 0)),
            pl.BlockSpec((1, cin), lambda i: (0, 0)),
        ],
        out_specs=[
            pl.BlockSpec((1, cin, FLAT), lambda i: (i, 0, 0)),
            pl.BlockSpec((cin + 1, cin), lambda i: (0, 0)),
        ],
        out_shape=[
            jax.ShapeDtypeStruct((BATCH, cin, FLAT), jnp.float32),
            jax.ShapeDtypeStruct((cin + 1, cin), jnp.float32),
        ],
    )(y, st, g.reshape(1, -1), be.reshape(1, -1))


def _final_body(h_ref, cov_ref, w_ref, bias_ref, g_ref, be_ref, out_ref):
    n = float(BATCH * FLAT)
    cov = cov_ref[...]
    m2 = cov[:-1] / n
    mu = (cov[-1] / n).reshape(-1, 1)
    w = w_ref[...]
    wm = jax.lax.dot_general(w, mu, (((1,), (0,)), ((), ())),
                             preferred_element_type=jnp.float32)
    t = jax.lax.dot_general(w, m2, (((1,), (0,)), ((), ())),
                            preferred_element_type=jnp.float32)
    e2 = jnp.sum(t * w, axis=1).reshape(-1, 1)
    var = e2 - wm * wm
    mean = wm + bias_ref[...].reshape(-1, 1)
    scale = g_ref[...].reshape(-1, 1) / jnp.sqrt(var + 1e-5)
    shift = be_ref[...].reshape(-1, 1) - mean * scale
    y = jax.lax.dot_general(w, h_ref[0], (((1,), (0,)), ((), ())),
                            preferred_element_type=jnp.float32)
    y = y + bias_ref[...].reshape(-1, 1)
    h = jnp.maximum(y * scale + shift, 0.0)
    out_ref[0] = jnp.max(h.reshape(-1, NCENT, NSAMP), axis=2)


def _final(h2, cov, w, bias, g, be, cin, cout):
    return pl.pallas_call(
        _final_body,
        grid=(BATCH,),
        in_specs=[
            pl.BlockSpec((1, cin, FLAT), lambda i: (i, 0, 0)),
            pl.BlockSpec((cin + 1, cin), lambda i: (0, 0)),
            pl.BlockSpec((cout, cin), lambda i: (0, 0)),
            pl.BlockSpec((1, cout), lambda i: (0, 0)),
            pl.BlockSpec((1, cout), lambda i: (0, 0)),
            pl.BlockSpec((1, cout), lambda i: (0, 0)),
        ],
        out_specs=pl.BlockSpec((1, cout, NCENT), lambda i: (i, 0, 0)),
        out_shape=jax.ShapeDtypeStruct((BATCH, cout, NCENT), jnp.float32),
    )(h2, cov, w, bias.reshape(1, -1), g.reshape(1, -1), be.reshape(1, -1))


def _bn_max_body(y_ref, st_ref, g_ref, be_ref, out_ref):
    st = st_ref[0]
    n = float(BATCH * FLAT)
    mu = st[0] / n
    var = st[1] / n - mu * mu
    scale = (g_ref[0] / jnp.sqrt(var + 1e-5)).reshape(-1, 1)
    shift = be_ref[0].reshape(-1, 1) - mu.reshape(-1, 1) * scale
    h = jnp.maximum(y_ref[0] * scale + shift, 0.0)
    out_ref[0] = jnp.max(h.reshape(-1, NCENT, NSAMP), axis=2)


def _bn_max(y, st, g, be, cin):
    cb = 32
    ng = cin // cb
    return pl.pallas_call(
        _bn_max_body,
        grid=(BATCH, ng),
        in_specs=[
            pl.BlockSpec((1, cb, FLAT), lambda i, j: (i, j, 0)),
            pl.BlockSpec((1, 2, cb), lambda i, j: (j, 0, 0)),
            pl.BlockSpec((1, 1, cb), lambda i, j: (j, 0, 0)),
            pl.BlockSpec((1, 1, cb), lambda i, j: (j, 0, 0)),
        ],
        out_specs=pl.BlockSpec((1, cb, NCENT), lambda i, j: (i, j, 0)),
        out_shape=jax.ShapeDtypeStruct((BATCH, cin, NCENT), jnp.float32),
    )(y, st.reshape(2, ng, cb).transpose(1, 0, 2),
      g.reshape(ng, 1, cb), be.reshape(ng, 1, cb))


def kernel(xyz, points, W0, b0, g0, be0, W1, b1, g1, be1, W2, b2, g2, be2):
    new_xyz = _fps(xyz)
    grouped = _group(xyz, points, new_xyz)
    y1, st1 = _conv_stats(grouped, W0, b0, 6, 64)
    y2, st2 = _bn_conv_stats(y1, st1, g0, be0, W1, b1, 64, 64)
    h2, cov2 = _bn_relu_cov(y2, st2, g1, be1, 64)
    x = _final(h2, cov2, W2, b2, g2, be2, 64, 128)
    return (new_xyz, x)


# speedup vs baseline: 24.5024x; 1.1653x over previous
import functools

import jax
import jax.numpy as jnp
from jax.experimental import pallas as pl
from jax.experimental.pallas import tpu as pltpu
from jax.experimental.pallas import tpu_sc as plsc

BATCH = 16
NPTS = 4096
NCENT = 512
NSAMP = 32
RAD2 = 0.2 * 0.2
NWORK = 32
ROWS_PER_W = BATCH * NCENT // NWORK
SHALF = NCENT // 2
NCHUNK = NPTS // 16
FLAT = NCENT * NSAMP


def _fps_body(xyz_ref, out_ref):
    x = xyz_ref[:, 0, :]
    y = xyz_ref[:, 1, :]
    z = xyz_ref[:, 2, :]
    iota = jax.lax.broadcasted_iota(jnp.int32, (BATCH, NPTS), 1)
    iota_c = jax.lax.broadcasted_iota(jnp.int32, (BATCH, NCENT), 1)

    def step(i, carry):
        dist, far, ax, ay, az = carry
        sel = iota == far
        cx = jnp.sum(jnp.where(sel, x, 0.0), axis=1, keepdims=True)
        cy = jnp.sum(jnp.where(sel, y, 0.0), axis=1, keepdims=True)
        cz = jnp.sum(jnp.where(sel, z, 0.0), axis=1, keepdims=True)
        here = iota_c == i
        ax = jnp.where(here, cx, ax)
        ay = jnp.where(here, cy, ay)
        az = jnp.where(here, cz, az)
        dx = x - cx
        dy = y - cy
        dz = z - cz
        d = dx * dx + dy * dy + dz * dz
        dist = jnp.minimum(dist, d)
        m = jnp.max(dist, axis=1, keepdims=True)
        far = jnp.min(jnp.where(dist == m, iota, NPTS), axis=1, keepdims=True)
        return dist, far, ax, ay, az

    zc = jnp.zeros((BATCH, NCENT), jnp.float32)
    init = (jnp.full((BATCH, NPTS), 1e10, jnp.float32),
            jnp.zeros((BATCH, 1), jnp.int32), zc, zc, zc)
    _, _, ax, ay, az = jax.lax.fori_loop(0, NCENT, step, init)
    out_ref[:, 0, :] = ax
    out_ref[:, 1, :] = ay
    out_ref[:, 2, :] = az


def _fps(xyz):
    return pl.pallas_call(
        _fps_body,
        out_shape=jax.ShapeDtypeStruct((BATCH, 3, NCENT), jnp.float32),
    )(xyz)


def _round_bf16(v):
    u = plsc.bitcast(v, jnp.uint32)
    r = (u + jnp.uint32(0x7FFF) + ((u >> jnp.uint32(16)) & jnp.uint32(1)))
    r = r & jnp.uint32(0xFFFF0000)
    return plsc.bitcast(r, jnp.float32)


def _group_body(xyz_hbm, pts_hbm, nxyz_hbm, out_hbm,
                xyz_v, pts_v, sq_v, bxyz_v, cent_v, idx_v, obuf_v):
    wid = jax.lax.axis_index("s") * 2 + jax.lax.axis_index("c")
    b = wid // 2
    s0 = (wid % 2) * SHALF

    pltpu.sync_copy(xyz_hbm.at[b], xyz_v)
    pltpu.sync_copy(pts_hbm.at[b], pts_v)
    for c in range(3):
        pltpu.sync_copy(nxyz_hbm.at[pl.ds((b * 3 + c) * NCENT + s0, SHALF)],
                        cent_v.at[pl.ds(c * SHALF, SHALF)])

    lane = jax.lax.iota(jnp.int32, 16)
    zeros16 = jnp.zeros((16,), jnp.int32)
    ones16 = jnp.full((16,), 1, jnp.int32)
    twos16 = jnp.full((16,), 2, jnp.int32)

    def sq_step(ci, _):
        xv = xyz_v[0, pl.ds(ci * 16, 16)]
        yv = xyz_v[1, pl.ds(ci * 16, 16)]
        zv = xyz_v[2, pl.ds(ci * 16, 16)]
        sq_v[pl.ds(ci * 16, 16)] = (xv * xv + yv * yv) + zv * zv
        bxyz_v[0, pl.ds(ci * 16, 16)] = _round_bf16(xv)
        bxyz_v[1, pl.ds(ci * 16, 16)] = _round_bf16(yv)
        bxyz_v[2, pl.ds(ci * 16, 16)] = _round_bf16(zv)
        return 0

    jax.lax.fori_loop(0, NCHUNK, sq_step, 0)

    def row_body(r, _):
        rv = jnp.full((16,), r, jnp.int32)
        cx = plsc.load_gather(cent_v, [rv])
        cy = plsc.load_gather(cent_v, [rv + SHALF])
        cz = plsc.load_gather(cent_v, [rv + 2 * SHALF])
        s2 = (cx * cx + cy * cy) + cz * cz
        bcx = _round_bf16(cx)
        bcy = _round_bf16(cy)
        bcz = _round_bf16(cz)

        def cond(st):
            g, found = st
            return jnp.logical_and(found < NSAMP, g < NCHUNK // 8)

        def body(st):
            g, found = st
            masks, csums, pcnts, gidxs = [], [], [], []
            tot = None
            for k in range(8):
                base = g * 128 + k * 16
                xv = bxyz_v[0, pl.ds(base, 16)]
                yv = bxyz_v[1, pl.ds(base, 16)]
                zv = bxyz_v[2, pl.ds(base, 16)]
                t = xv * bcx + yv * bcy
                t = t + zv * bcz
                d = (-2.0) * t + s2
                d = d + sq_v[pl.ds(base, 16)]
                mask = d <= RAD2
                mi = mask.astype(jnp.int32)
                masks.append(mask)
                csums.append(jnp.cumsum(mi))
                pcnts.append(plsc.all_reduce_population_count(mask))
                gidxs.append(base + lane)
                tot = mi if tot is None else tot + mi
            total = jnp.sum(tot)

            @pl.when(jnp.logical_and(found == 0, total > 0))
            def _prefill():
                fs = [jnp.where(masks[k], gidxs[k], NPTS) for k in range(8)]
                m0 = jnp.minimum(jnp.minimum(fs[0], fs[1]),
                                 jnp.minimum(fs[2], fs[3]))
                m1 = jnp.minimum(jnp.minimum(fs[4], fs[5]),
                                 jnp.minimum(fs[6], fs[7]))
                first = jnp.min(jnp.minimum(m0, m1))
                fill = jnp.full((16,), first, jnp.int32)
                idx_v[pl.ds(0, 16)] = fill
                idx_v[pl.ds(16, 16)] = fill

            found_vv = jnp.full((16,), found, jnp.int32)
            off = found_vv
            for k in range(8):
                pos = off + csums[k] - 1
                wmask = jnp.logical_and(masks[k], pos < NSAMP)
                plsc.store_scatter(idx_v, [pos], gidxs[k], mask=wmask)
                off = off + pcnts[k]
            return g + 1, found + total

        jax.lax.while_loop(cond, body, (jnp.int32(0), jnp.int32(0)))

        plane = SHALF * NSAMP
        for half in range(2):
            iv = idx_v[pl.ds(half * 16, 16)]
            col = r * NSAMP + half * 16
            gx = plsc.load_gather(xyz_v, [zeros16, iv])
            gy = plsc.load_gather(xyz_v, [ones16, iv])
            gz = plsc.load_gather(xyz_v, [twos16, iv])
            obuf_v[pl.ds(col, 16)] = gx - cx
            obuf_v[pl.ds(plane + col, 16)] = gy - cy
            obuf_v[pl.ds(2 * plane + col, 16)] = gz - cz
            obuf_v[pl.ds(3 * plane + col, 16)] = plsc.load_gather(
                pts_v, [zeros16, iv])
            obuf_v[pl.ds(4 * plane + col, 16)] = plsc.load_gather(
                pts_v, [ones16, iv])
            obuf_v[pl.ds(5 * plane + col, 16)] = plsc.load_gather(
                pts_v, [twos16, iv])
        return 0

    jax.lax.fori_loop(0, SHALF, row_body, 0)

    plane = SHALF * NSAMP
    for c in range(6):
        pltpu.sync_copy(
            obuf_v.at[pl.ds(c * plane, plane)],
            out_hbm.at[pl.ds((b * 6 + c) * FLAT + s0 * NSAMP, plane)])


def _group(xyz, points, new_xyz):
    mesh = plsc.VectorSubcoreMesh(core_axis_name="c", subcore_axis_name="s")
    f = functools.partial(
        pl.kernel,
        mesh=mesh,
        compiler_params=pltpu.CompilerParams(needs_layout_passes=False),
        out_type=jax.ShapeDtypeStruct((BATCH * 6 * FLAT,), jnp.float32),
        scratch_types=[
            pltpu.VMEM((3, NPTS), jnp.float32),
            pltpu.VMEM((3, NPTS), jnp.float32),
            pltpu.VMEM((NPTS,), jnp.float32),
            pltpu.VMEM((3, NPTS), jnp.float32),
            pltpu.VMEM((3 * SHALF,), jnp.float32),
            pltpu.VMEM((NSAMP,), jnp.int32),
            pltpu.VMEM((6 * SHALF * NSAMP,), jnp.float32),
        ],
    )(_group_body)
    return f(xyz, points, new_xyz.reshape(-1)).reshape(BATCH, 6, FLAT)


def _conv_stats_body(x_ref, w_ref, bias_ref, st_ref):
    i = pl.program_id(0)
    y = jax.lax.dot_general(w_ref[...], x_ref[0],
                            (((1,), (0,)), ((), ())),
                            preferred_element_type=jnp.float32)
    y = y + bias_ref[...].reshape(-1, 1)
    s1 = jnp.sum(y, axis=1)
    s2 = jnp.sum(y * y, axis=1)
    st = jnp.stack([s1, s2])

    @pl.when(i == 0)
    def _():
        st_ref[...] = st

    @pl.when(i > 0)
    def _():
        st_ref[...] = st_ref[...] + st


def _conv_stats(x, w, bias, cin, cout):
    return pl.pallas_call(
        _conv_stats_body,
        grid=(BATCH,),
        in_specs=[
            pl.BlockSpec((1, cin, FLAT), lambda i: (i, 0, 0)),
            pl.BlockSpec((cout, cin), lambda i: (0, 0)),
            pl.BlockSpec((1, cout), lambda i: (0, 0)),
        ],
        out_specs=pl.BlockSpec((2, cout), lambda i: (0, 0)),
        out_shape=jax.ShapeDtypeStruct((2, cout), jnp.float32),
    )(x, w, bias.reshape(1, -1))


def _bn_params(st, g, be):
    n = float(BATCH * FLAT)
    mu = st[0] / n
    var = st[1] / n - mu * mu
    scale = g / jnp.sqrt(var + 1e-5)
    shift = be - mu * scale
    return scale, shift


def _conv_bn_conv_body(x_ref, st_ref, w0_ref, b0_ref, g_ref, be_ref,
                       w_ref, bias_ref, y2_ref, st2_ref):
    i = pl.program_id(0)
    y1 = jax.lax.dot_general(w0_ref[...], x_ref[0],
                             (((1,), (0,)), ((), ())),
                             preferred_element_type=jnp.float32)
    y1 = y1 + b0_ref[...].reshape(-1, 1)
    st = st_ref[...]
    n = float(BATCH * FLAT)
    mu = st[0] / n
    var = st[1] / n - mu * mu
    scale = (g_ref[...] / jnp.sqrt(var + 1e-5)).reshape(-1, 1)
    shift = be_ref[...].reshape(-1, 1) - mu.reshape(-1, 1) * scale
    h = jnp.maximum(y1 * scale + shift, 0.0)
    y2 = jax.lax.dot_general(w_ref[...], h,
                             (((1,), (0,)), ((), ())),
                             preferred_element_type=jnp.float32)
    y2 = y2 + bias_ref[...].reshape(-1, 1)
    y2_ref[0] = y2
    s1 = jnp.sum(y2, axis=1)
    s2 = jnp.sum(y2 * y2, axis=1)
    stn = jnp.stack([s1, s2])

    @pl.when(i == 0)
    def _():
        st2_ref[...] = stn

    @pl.when(i > 0)
    def _():
        st2_ref[...] = st2_ref[...] + stn


def _conv_bn_conv(x, st, w0, b0, g, be, w, bias, cin, cmid, cout):
    return pl.pallas_call(
        _conv_bn_conv_body,
        grid=(BATCH,),
        in_specs=[
            pl.BlockSpec((1, cin, FLAT), lambda i: (i, 0, 0)),
            pl.BlockSpec((2, cmid), lambda i: (0, 0)),
            pl.BlockSpec((cmid, cin), lambda i: (0, 0)),
            pl.BlockSpec((1, cmid), lambda i: (0, 0)),
            pl.BlockSpec((1, cmid), lambda i: (0, 0)),
            pl.BlockSpec((1, cmid), lambda i: (0, 0)),
            pl.BlockSpec((cout, cmid), lambda i: (0, 0)),
            pl.BlockSpec((1, cout), lambda i: (0, 0)),
        ],
        out_specs=[
            pl.BlockSpec((1, cout, FLAT), lambda i: (i, 0, 0)),
            pl.BlockSpec((2, cout), lambda i: (0, 0)),
        ],
        out_shape=[
            jax.ShapeDtypeStruct((BATCH, cout, FLAT), jnp.float32),
            jax.ShapeDtypeStruct((2, cout), jnp.float32),
        ],
    )(x, st, w0, b0.reshape(1, -1), g.reshape(1, -1), be.reshape(1, -1),
      w, bias.reshape(1, -1))


def _bn_relu_cov_body(y_ref, st_ref, g_ref, be_ref, h_ref, cov_ref):
    i = pl.program_id(0)
    st = st_ref[...]
    n = float(BATCH * FLAT)
    mu = st[0] / n
    var = st[1] / n - mu * mu
    scale = (g_ref[...] / jnp.sqrt(var + 1e-5)).reshape(-1, 1)
    shift = be_ref[...].reshape(-1, 1) - mu.reshape(-1, 1) * scale
    h = jnp.maximum(y_ref[0] * scale + shift, 0.0)
    h_ref[0] = h
    m2 = jax.lax.dot_general(h, h, (((1,), (1,)), ((), ())),
                             preferred_element_type=jnp.float32)
    s = jnp.sum(h, axis=1).reshape(1, -1)
    cov = jnp.concatenate([m2, s], axis=0)

    @pl.when(i == 0)
    def _():
        cov_ref[...] = cov

    @pl.when(i > 0)
    def _():
        cov_ref[...] = cov_ref[...] + cov


def _bn_relu_cov(y, st, g, be, cin):
    return pl.pallas_call(
        _bn_relu_cov_body,
        grid=(BATCH,),
        in_specs=[
            pl.BlockSpec((1, cin, FLAT), lambda i: (i, 0, 0)),
            pl.BlockSpec((2, cin), lambda i: (0, 0)),
            pl.BlockSpec((1, cin), lambda i: (0, 0)),
            pl.BlockSpec((1, cin), lambda i: (0, 0)),
        ],
        out_specs=[
            pl.BlockSpec((1, cin, FLAT), lambda i: (i, 0, 0)),
            pl.BlockSpec((cin + 1, cin), lambda i: (0, 0)),
        ],
        out_shape=[
            jax.ShapeDtypeStruct((BATCH, cin, FLAT), jnp.float32),
            jax.ShapeDtypeStruct((cin + 1, cin), jnp.float32),
        ],
    )(y, st, g.reshape(1, -1), be.reshape(1, -1))


def _final_body(h_ref, cov_ref, w_ref, bias_ref, g_ref, be_ref, out_ref):
    n = float(BATCH * FLAT)
    cov = cov_ref[...]
    m2 = cov[:-1] / n
    mu = (cov[-1] / n).reshape(-1, 1)
    w = w_ref[...]
    wm = jax.lax.dot_general(w, mu, (((1,), (0,)), ((), ())),
                             preferred_element_type=jnp.float32)
    t = jax.lax.dot_general(w, m2, (((1,), (0,)), ((), ())),
                            preferred_element_type=jnp.float32)
    e2 = jnp.sum(t * w, axis=1).reshape(-1, 1)
    var = e2 - wm * wm
    mean = wm + bias_ref[...].reshape(-1, 1)
    scale = g_ref[...].reshape(-1, 1) / jnp.sqrt(var + 1e-5)
    shift = be_ref[...].reshape(-1, 1) - mean * scale
    y = jax.lax.dot_general(w, h_ref[0], (((1,), (0,)), ((), ())),
                            preferred_element_type=jnp.float32)
    y = y + bias_ref[...].reshape(-1, 1)
    h = jnp.maximum(y * scale + shift, 0.0)
    out_ref[0] = jnp.max(h.reshape(-1, NCENT, NSAMP), axis=2)


def _final(h2, cov, w, bias, g, be, cin, cout):
    return pl.pallas_call(
        _final_body,
        grid=(BATCH,),
        in_specs=[
            pl.BlockSpec((1, cin, FLAT), lambda i: (i, 0, 0)),
            pl.BlockSpec((cin + 1, cin), lambda i: (0, 0)),
            pl.BlockSpec((cout, cin), lambda i: (0, 0)),
            pl.BlockSpec((1, cout), lambda i: (0, 0)),
            pl.BlockSpec((1, cout), lambda i: (0, 0)),
            pl.BlockSpec((1, cout), lambda i: (0, 0)),
        ],
        out_specs=pl.BlockSpec((1, cout, NCENT), lambda i: (i, 0, 0)),
        out_shape=jax.ShapeDtypeStruct((BATCH, cout, NCENT), jnp.float32),
    )(h2, cov, w, bias.reshape(1, -1), g.reshape(1, -1), be.reshape(1, -1))


def _bn_max_body(y_ref, st_ref, g_ref, be_ref, out_ref):
    st = st_ref[0]
    n = float(BATCH * FLAT)
    mu = st[0] / n
    var = st[1] / n - mu * mu
    scale = (g_ref[0] / jnp.sqrt(var + 1e-5)).reshape(-1, 1)
    shift = be_ref[0].reshape(-1, 1) - mu.reshape(-1, 1) * scale
    h = jnp.maximum(y_ref[0] * scale + shift, 0.0)
    out_ref[0] = jnp.max(h.reshape(-1, NCENT, NSAMP), axis=2)


def _bn_max(y, st, g, be, cin):
    cb = 32
    ng = cin // cb
    return pl.pallas_call(
        _bn_max_body,
        grid=(BATCH, ng),
        in_specs=[
            pl.BlockSpec((1, cb, FLAT), lambda i, j: (i, j, 0)),
            pl.BlockSpec((1, 2, cb), lambda i, j: (j, 0, 0)),
            pl.BlockSpec((1, 1, cb), lambda i, j: (j, 0, 0)),
            pl.BlockSpec((1, 1, cb), lambda i, j: (j, 0, 0)),
        ],
        out_specs=pl.BlockSpec((1, cb, NCENT), lambda i, j: (i, j, 0)),
        out_shape=jax.ShapeDtypeStruct((BATCH, cin, NCENT), jnp.float32),
    )(y, st.reshape(2, ng, cb).transpose(1, 0, 2),
      g.reshape(ng, 1, cb), be.reshape(ng, 1, cb))


def kernel(xyz, points, W0, b0, g0, be0, W1, b1, g1, be1, W2, b2, g2, be2):
    new_xyz = _fps(xyz)
    grouped = _group(xyz, points, new_xyz)
    st1 = _conv_stats(grouped, W0, b0, 6, 64)
    y2, st2 = _conv_bn_conv(grouped, st1, W0, b0, g0, be0, W1, b1, 6, 64, 64)
    h2, cov2 = _bn_relu_cov(y2, st2, g1, be1, 64)
    x = _final(h2, cov2, W2, b2, g2, be2, 64, 128)
    return (new_xyz, x)
